# Initial kernel scaffold; baseline (speedup 1.0000x reference)
import jax
import jax.numpy as jnp
from jax.experimental import pallas as pl


def kernel(w_id, table):
    raise NotImplementedError("write your pallas kernel here")


# speedup vs baseline: 6.3280x; 6.3280x over previous
import functools

import jax
import jax.numpy as jnp
from jax import lax
from jax.experimental import pallas as pl
from jax.experimental.pallas import tpu as pltpu
from jax.experimental.pallas import tpu_sc as plsc

VOCAB = 100000
D = 128
B = 4096
L = 200
TOT = B * L
NC = 2
NS = 16
NW = NC * NS
PER_W = TOT // NW
CHUNK = 128
NCH = PER_W // CHUNK

_mesh = plsc.VectorSubcoreMesh(core_axis_name="c", subcore_axis_name="s")


@functools.partial(
    pl.kernel,
    mesh=_mesh,
    out_type=jax.ShapeDtypeStruct((TOT, D), jnp.float32),
    scratch_types=[
        pltpu.VMEM((NCH, CHUNK), jnp.int32),
        pltpu.VMEM((CHUNK, D), jnp.float32),
        pltpu.SemaphoreType.DMA,
    ],
)
def _embed_gather(idx_hbm, table_hbm, out_hbm, idx_v, rows_v, sem):
    wid = lax.axis_index("s") * NC + lax.axis_index("c")
    pltpu.sync_copy(idx_hbm.at[wid], idx_v)
    base = wid * PER_W

    def body(j, carry):
        pltpu.async_copy(table_hbm.at[idx_v.at[j]], rows_v, sem).wait()
        pltpu.sync_copy(rows_v, out_hbm.at[pl.ds(base + j * CHUNK, CHUNK)])
        return carry

    lax.fori_loop(0, NCH, body, 0)


def kernel(w_id, table):
    idx = w_id.astype(jnp.int32).reshape(NW, NCH, CHUNK)
    out = _embed_gather(idx, table)
    return out.reshape(B, L, D)


# speedup vs baseline: 9.2409x; 1.4603x over previous
import functools

import jax
import jax.numpy as jnp
from jax import lax
from jax.experimental import pallas as pl
from jax.experimental.pallas import tpu as pltpu
from jax.experimental.pallas import tpu_sc as plsc

VOCAB = 100000
D = 128
B = 4096
L = 200
TOT = B * L
NC = 2
NS = 16
NW = NC * NS
PER_W = TOT // NW
CHUNK = 128
NCH = PER_W // CHUNK
NBUF = 4

_mesh = plsc.VectorSubcoreMesh(core_axis_name="c", subcore_axis_name="s")


@functools.partial(
    pl.kernel,
    mesh=_mesh,
    out_type=jax.ShapeDtypeStruct((TOT, D), jnp.float32),
    scratch_types=[
        pltpu.VMEM((NCH, CHUNK), jnp.int32),
        pltpu.VMEM((NBUF, CHUNK, D), jnp.float32),
    ] + [pltpu.SemaphoreType.DMA] * (2 * NBUF),
)
def _embed_gather(idx_hbm, table_hbm, out_hbm, idx_v, rows_v, *sems):
    gsems, ssems = sems[:NBUF], sems[NBUF:]
    wid = lax.axis_index("s") * NC + lax.axis_index("c")
    pltpu.sync_copy(idx_hbm.at[wid], idx_v)
    base = wid * PER_W

    def start_gather(j, b):
        pltpu.async_copy(table_hbm.at[idx_v.at[j]], rows_v.at[b], gsems[b])

    def wait_gather(b):
        pltpu.make_async_copy(
            table_hbm.at[pl.ds(0, CHUNK)], rows_v.at[b], gsems[b]
        ).wait()

    def out_slab(j):
        return out_hbm.at[pl.ds(base + j * CHUNK, CHUNK)]

    for b in range(NBUF):
        start_gather(b, b)

    def round_body(i, carry):
        j0 = i * NBUF
        for b in range(NBUF):
            j = j0 + b
            wait_gather(b)
            pltpu.async_copy(rows_v.at[b], out_slab(j), ssems[b])
            jn = j + NBUF

            @pl.when(jn < NCH)
            def _():
                pltpu.make_async_copy(rows_v.at[b], out_slab(j), ssems[b]).wait()
                start_gather(jn, b)
        return carry

    lax.fori_loop(0, NCH // NBUF, round_body, 0)

    for b in range(NBUF):
        j = NCH - NBUF + b
        pltpu.make_async_copy(rows_v.at[b], out_slab(j), ssems[b]).wait()


def kernel(w_id, table):
    idx = w_id.astype(jnp.int32).reshape(NW, NCH, CHUNK)
    out = _embed_gather(idx, table)
    return out.reshape(B, L, D)


# speedup vs baseline: 9.2495x; 1.0009x over previous
import functools

import jax
import jax.numpy as jnp
from jax import lax
from jax.experimental import pallas as pl
from jax.experimental.pallas import tpu as pltpu
from jax.experimental.pallas import tpu_sc as plsc

VOCAB = 100000
D = 128
B = 4096
L = 200
TOT = B * L
NC = 2
NS = 16
NW = NC * NS
PER_W = TOT // NW
CHUNK = 128
NCH = PER_W // CHUNK
NBUF = 5

_mesh = plsc.VectorSubcoreMesh(core_axis_name="c", subcore_axis_name="s")


@functools.partial(
    pl.kernel,
    mesh=_mesh,
    out_type=jax.ShapeDtypeStruct((TOT, D), jnp.float32),
    scratch_types=[
        pltpu.VMEM((NCH, CHUNK), jnp.int32),
        pltpu.VMEM((NBUF, CHUNK, D), jnp.float32),
    ] + [pltpu.SemaphoreType.DMA] * (2 * NBUF),
)
def _embed_gather(idx_hbm, table_hbm, out_hbm, idx_v, rows_v, *sems):
    gsems, ssems = sems[:NBUF], sems[NBUF:]
    wid = lax.axis_index("s") * NC + lax.axis_index("c")
    pltpu.sync_copy(idx_hbm.at[wid], idx_v)
    base = wid * PER_W

    def start_gather(j, b):
        pltpu.async_copy(table_hbm.at[idx_v.at[j]], rows_v.at[b], gsems[b])

    def wait_gather(b):
        pltpu.make_async_copy(
            table_hbm.at[pl.ds(0, CHUNK)], rows_v.at[b], gsems[b]
        ).wait()

    def out_slab(j):
        return out_hbm.at[pl.ds(base + j * CHUNK, CHUNK)]

    for b in range(NBUF):
        start_gather(b, b)

    def round_body(i, carry):
        j0 = i * NBUF
        for b in range(NBUF):
            j = j0 + b
            wait_gather(b)
            pltpu.async_copy(rows_v.at[b], out_slab(j), ssems[b])
            jn = j + NBUF

            @pl.when(jn < NCH)
            def _():
                pltpu.make_async_copy(rows_v.at[b], out_slab(j), ssems[b]).wait()
                start_gather(jn, b)
        return carry

    lax.fori_loop(0, NCH // NBUF, round_body, 0)

    for b in range(NBUF):
        j = NCH - NBUF + b
        pltpu.make_async_copy(rows_v.at[b], out_slab(j), ssems[b]).wait()


def kernel(w_id, table):
    idx = w_id.astype(jnp.int32).reshape(NW, NCH, CHUNK)
    out = _embed_gather(idx, table)
    return out.reshape(B, L, D)


# speedup vs baseline: 9.2545x; 1.0005x over previous
import functools

import jax
import jax.numpy as jnp
from jax import lax
from jax.experimental import pallas as pl
from jax.experimental.pallas import tpu as pltpu
from jax.experimental.pallas import tpu_sc as plsc

VOCAB = 100000
D = 128
B = 4096
L = 200
TOT = B * L
NC = 2
NS = 16
NW = NC * NS
PER_W = TOT // NW
CHUNK = 128
NCH = PER_W // CHUNK
NBUF = 5
LOOKAHEAD = 3

_mesh = plsc.VectorSubcoreMesh(core_axis_name="c", subcore_axis_name="s")


@functools.partial(
    pl.kernel,
    mesh=_mesh,
    out_type=jax.ShapeDtypeStruct((TOT, D), jnp.float32),
    scratch_types=[
        pltpu.VMEM((NCH, CHUNK), jnp.int32),
        pltpu.VMEM((NBUF, CHUNK, D), jnp.float32),
    ] + [pltpu.SemaphoreType.DMA] * (2 * NBUF),
)
def _embed_gather(idx_hbm, table_hbm, out_hbm, idx_v, rows_v, *sems):
    gsems, ssems = sems[:NBUF], sems[NBUF:]
    wid = lax.axis_index("s") * NC + lax.axis_index("c")
    pltpu.sync_copy(idx_hbm.at[wid], idx_v)
    base = wid * PER_W

    def start_gather(j, b):
        pltpu.async_copy(table_hbm.at[idx_v.at[j]], rows_v.at[b], gsems[b])

    def wait_gather(b):
        pltpu.make_async_copy(
            table_hbm.at[pl.ds(0, CHUNK)], rows_v.at[b], gsems[b]
        ).wait()

    def out_slab(j):
        return out_hbm.at[pl.ds(base + j * CHUNK, CHUNK)]

    for b in range(LOOKAHEAD):
        start_gather(b, b)

    def round_body(i, carry):
        j0 = i * NBUF
        for b in range(NBUF):
            j = j0 + b
            wait_gather(b)
            pltpu.async_copy(rows_v.at[b], out_slab(j), ssems[b])
            jn = j + LOOKAHEAD
            bn = (b + LOOKAHEAD) % NBUF

            @pl.when(jn < NCH)
            def _():
                @pl.when(jn >= NBUF)
                def _():
                    pltpu.make_async_copy(
                        rows_v.at[bn], out_slab(jn - NBUF), ssems[bn]
                    ).wait()

                start_gather(jn, bn)
        return carry

    lax.fori_loop(0, NCH // NBUF, round_body, 0)

    for b in range(NBUF):
        j = NCH - NBUF + b
        pltpu.make_async_copy(rows_v.at[b], out_slab(j), ssems[b]).wait()


def kernel(w_id, table):
    idx = w_id.astype(jnp.int32).reshape(NW, NCH, CHUNK)
    out = _embed_gather(idx, table)
    return out.reshape(B, L, D)
